# baseline (speedup 1.0000x reference)
import jax
import jax.numpy as jnp
from jax.experimental import pallas as pl
from jax.experimental.pallas import tpu as pltpu

NUM_SAMPLE = 9
ALPHA = 0.25
GAMMA = 2.0
BLK_P = 512


def _cost_topk_kernel(lt_ref, lp_ref, pp_ref, pt_ref, mp_ref, mt_ref,
                      pct_ref, pi_ref, c_ref):
    i = pl.program_id(1)
    nblk = pl.num_programs(1)
    g = lt_ref.shape[2]
    nray = pp_ref.shape[2] // g
    hw = mp_ref.shape[2]

    lt = lt_ref[0]
    labels_row = lt[0:1, :]

    x = lp_ref[0]
    lp = jax.nn.sigmoid(x)
    neg = (1.0 - ALPHA) * lp ** GAMMA * -jnp.log(1.0 - lp + 1e-08)
    pos = ALPHA * (1.0 - lp) ** GAMMA * -jnp.log(lp + 1e-08)
    diff = pos - neg
    ncls = x.shape[1]
    onehot = (jax.lax.broadcasted_iota(jnp.int32, (ncls, g), 0)
              == labels_row).astype(jnp.float32)
    cc = jnp.dot(diff, onehot, preferred_element_type=jnp.float32, precision=jax.lax.Precision.HIGHEST)

    ppv = pp_ref[0]
    ptv = pt_ref[0]
    lmax = jnp.maximum(ppv, ptv)
    lmin = jnp.minimum(ppv, ptv)
    segray = (jax.lax.broadcasted_iota(jnp.int32, (g * nray, g), 0) // nray
              == jax.lax.broadcasted_iota(jnp.int32, (g * nray, g), 1)
              ).astype(jnp.float32)
    smax = jnp.dot(lmax, segray, preferred_element_type=jnp.float32, precision=jax.lax.Precision.HIGHEST)
    smin = jnp.dot(lmin, segray, preferred_element_type=jnp.float32, precision=jax.lax.Precision.HIGHEST)
    vm = jnp.log(smax / smin)

    mpv = mp_ref[0]
    mtv = mt_ref[0]
    hw2 = hw // 2
    prod = mpv * mtv
    summ = mpv + mtv
    mcat = jnp.concatenate([prod[:, :hw2] + prod[:, hw2:],
                            summ[:, :hw2] + summ[:, hw2:]], axis=1)
    iotm = jax.lax.broadcasted_iota(jnp.int32, (hw, 2), 0)
    selm = (iotm // hw2
            == jax.lax.broadcasted_iota(jnp.int32, (hw, 2), 1)
            ).astype(jnp.float32)
    ab = jnp.dot(mcat, selm, preferred_element_type=jnp.float32, precision=jax.lax.Precision.HIGHEST)
    dice = (2.0 * ab[:, 0:1] + 1.0) / (ab[:, 1:2] + 1.0)

    c_rows = 1.0 - dice
    blk_p = lp_ref.shape[1]
    c_ref[pl.ds(i * blk_p, blk_p), :] = c_rows.reshape(blk_p, g) + vm + cc

    @pl.when(i == nblk - 1)
    def _():
        c = c_ref[:, :]
        p = c.shape[0]
        iota_r = jax.lax.broadcasted_iota(jnp.int32, (p, g), 0)
        iota_c = jax.lax.broadcasted_iota(jnp.int32, (p, g), 1)
        cols8 = jax.lax.broadcasted_iota(jnp.int32, (1, g), 1)
        best = jnp.full((p, g), -1, jnp.int32)
        pi_rows = []
        for k in range(NUM_SAMPLE):
            m = jnp.min(c, axis=0, keepdims=True)
            idxk = jnp.min(jnp.where(c == m, iota_r, p),
                           axis=0, keepdims=True)
            match = iota_r == idxk
            enc = jnp.where(match, (k * g + iota_c) * 128 + labels_row, -1)
            best = jnp.maximum(best, enc)
            pi_rows.append(idxk * g + cols8)
            c = jnp.where(match, jnp.float32(jnp.inf), c)
        best1 = jnp.max(best, axis=1, keepdims=True)
        pct_ref[0] = jnp.where(best1 < 0, 80,
                               jnp.bitwise_and(best1, 127)).astype(jnp.int32)
        pi_rows += [jnp.zeros((1, g), jnp.int32)] * (16 - NUM_SAMPLE)
        pi_ref[0] = jnp.concatenate(pi_rows, axis=0)


def kernel(label_targs, label_preds, poly_targs, poly_preds,
           mask_targs, mask_preds, inside_indices):
    b, p, _ = label_preds.shape
    g = label_targs.shape[1]
    nray = poly_targs.shape[-1]
    hw = mask_targs.shape[-1]
    nblk = p // BLK_P

    lt3 = jnp.broadcast_to(label_targs[:, None, :].astype(jnp.int32),
                           (b, 8, g))
    blk_ni = BLK_P * g

    pct3, pi3 = pl.pallas_call(
        _cost_topk_kernel,
        grid=(b, nblk),
        in_specs=[
            pl.BlockSpec((1, 8, g), lambda bi, i: (bi, 0, 0)),
            pl.BlockSpec((1, BLK_P, 80), lambda bi, i: (bi, i, 0)),
            pl.BlockSpec((1, BLK_P, g * nray), lambda bi, i: (bi, i, 0)),
            pl.BlockSpec((1, BLK_P, g * nray), lambda bi, i: (bi, i, 0)),
            pl.BlockSpec((1, blk_ni, hw), lambda bi, i: (bi, i, 0)),
            pl.BlockSpec((1, blk_ni, hw), lambda bi, i: (bi, i, 0)),
        ],
        out_specs=[
            pl.BlockSpec((1, p, 1), lambda bi, i: (bi, 0, 0)),
            pl.BlockSpec((1, 16, g), lambda bi, i: (bi, 0, 0)),
        ],
        out_shape=[
            jax.ShapeDtypeStruct((b, p, 1), jnp.int32),
            jax.ShapeDtypeStruct((b, 16, g), jnp.int32),
        ],
        scratch_shapes=[pltpu.VMEM((p, g), jnp.float32)],
        compiler_params=pltpu.CompilerParams(
            dimension_semantics=("arbitrary", "arbitrary")),
    )(lt3, label_preds,
      poly_preds.reshape(b, p, g * nray), poly_targs.reshape(b, p, g * nray),
      mask_preds, mask_targs)

    pos_class_targ = pct3[:, :, 0]
    pos_indices = pi3[:, :NUM_SAMPLE, :].reshape(b, NUM_SAMPLE * g)
    return pos_class_targ, pos_indices


# speedup vs baseline: 1.0395x; 1.0395x over previous
import jax
import jax.numpy as jnp
from jax.experimental import pallas as pl
from jax.experimental.pallas import tpu as pltpu

NUM_SAMPLE = 9
ALPHA = 0.25
GAMMA = 2.0
BLK_P = 512


def _class_mask_kernel(lt_ref, lp_ref, mp_ref, mt_ref, cpart_ref):
    g = lt_ref.shape[2]
    hw = mp_ref.shape[2]

    lt = lt_ref[0]
    labels_row = lt[0:1, :]

    x = lp_ref[0]
    lp = jax.nn.sigmoid(x)
    neg = (1.0 - ALPHA) * lp ** GAMMA * -jnp.log(1.0 - lp + 1e-08)
    pos = ALPHA * (1.0 - lp) ** GAMMA * -jnp.log(lp + 1e-08)
    diff = pos - neg
    ncls = x.shape[1]
    onehot = (jax.lax.broadcasted_iota(jnp.int32, (ncls, g), 0)
              == labels_row).astype(jnp.float32)
    cc = jnp.dot(diff, onehot, preferred_element_type=jnp.float32,
                 precision=jax.lax.Precision.HIGHEST)

    mpv = mp_ref[0]
    mtv = mt_ref[0]
    hw2 = hw // 2
    prod = mpv * mtv
    summ = mpv + mtv
    mcat = jnp.concatenate([prod[:, :hw2] + prod[:, hw2:],
                            summ[:, :hw2] + summ[:, hw2:]], axis=1)
    iotm = jax.lax.broadcasted_iota(jnp.int32, (hw, 2), 0)
    selm = (iotm // hw2
            == jax.lax.broadcasted_iota(jnp.int32, (hw, 2), 1)
            ).astype(jnp.float32)
    ab = jnp.dot(mcat, selm, preferred_element_type=jnp.float32,
                 precision=jax.lax.Precision.HIGHEST)
    dice = (2.0 * ab[:, 0:1] + 1.0) / (ab[:, 1:2] + 1.0)

    blk_p = lp_ref.shape[1]
    cpart_ref[0] = (1.0 - dice).reshape(blk_p, g) + cc


def _poly_topk_kernel(lt_ref, cp_ref, pp_ref, pt_ref,
                      pct_ref, pi_ref, c_ref):
    i = pl.program_id(1)
    nblk = pl.num_programs(1)
    g = lt_ref.shape[2]
    nray = pp_ref.shape[2] // g

    lt = lt_ref[0]
    labels_row = lt[0:1, :]

    ppv = pp_ref[0]
    ptv = pt_ref[0]
    lmax = jnp.maximum(ppv, ptv)
    lmin = jnp.minimum(ppv, ptv)
    segray = (jax.lax.broadcasted_iota(jnp.int32, (g * nray, g), 0) // nray
              == jax.lax.broadcasted_iota(jnp.int32, (g * nray, g), 1)
              ).astype(jnp.float32)
    smax = jnp.dot(lmax, segray, preferred_element_type=jnp.float32,
                   precision=jax.lax.Precision.HIGHEST)
    smin = jnp.dot(lmin, segray, preferred_element_type=jnp.float32,
                   precision=jax.lax.Precision.HIGHEST)
    vm = jnp.log(smax / smin)

    blk_p = cp_ref.shape[1]
    c_ref[pl.ds(i * blk_p, blk_p), :] = cp_ref[0] + vm

    @pl.when(i == nblk - 1)
    def _():
        c = c_ref[:, :]
        p = c.shape[0]
        iota_r = jax.lax.broadcasted_iota(jnp.int32, (p, g), 0)
        iota_c = jax.lax.broadcasted_iota(jnp.int32, (p, g), 1)
        cols8 = jax.lax.broadcasted_iota(jnp.int32, (1, g), 1)
        best = jnp.full((p, g), -1, jnp.int32)
        pi_rows = []
        for k in range(NUM_SAMPLE):
            m = jnp.min(c, axis=0, keepdims=True)
            idxk = jnp.min(jnp.where(c == m, iota_r, p),
                           axis=0, keepdims=True)
            match = iota_r == idxk
            enc = jnp.where(match, (k * g + iota_c) * 128 + labels_row, -1)
            best = jnp.maximum(best, enc)
            pi_rows.append(idxk * g + cols8)
            c = jnp.where(match, jnp.float32(jnp.inf), c)
        best1 = jnp.max(best, axis=1, keepdims=True)
        pct_ref[0] = jnp.where(best1 < 0, 80,
                               jnp.bitwise_and(best1, 127)).astype(jnp.int32)
        pi_rows += [jnp.zeros((1, g), jnp.int32)] * (16 - NUM_SAMPLE)
        pi_ref[0] = jnp.concatenate(pi_rows, axis=0)


def kernel(label_targs, label_preds, poly_targs, poly_preds,
           mask_targs, mask_preds, inside_indices):
    b, p, _ = label_preds.shape
    g = label_targs.shape[1]
    nray = poly_targs.shape[-1]
    hw = mask_targs.shape[-1]
    nblk = p // BLK_P
    blk_ni = BLK_P * g

    lt3 = jnp.broadcast_to(label_targs[:, None, :].astype(jnp.int32),
                           (b, 8, g))

    cpart = pl.pallas_call(
        _class_mask_kernel,
        grid=(b, nblk),
        in_specs=[
            pl.BlockSpec((1, 8, g), lambda bi, i: (bi, 0, 0)),
            pl.BlockSpec((1, BLK_P, 80), lambda bi, i: (bi, i, 0)),
            pl.BlockSpec((1, blk_ni, hw), lambda bi, i: (bi, i, 0)),
            pl.BlockSpec((1, blk_ni, hw), lambda bi, i: (bi, i, 0)),
        ],
        out_specs=pl.BlockSpec((1, BLK_P, g), lambda bi, i: (bi, i, 0)),
        out_shape=jax.ShapeDtypeStruct((b, p, g), jnp.float32),
        compiler_params=pltpu.CompilerParams(
            dimension_semantics=("arbitrary", "arbitrary")),
    )(lt3, label_preds, mask_preds, mask_targs)

    pct3, pi3 = pl.pallas_call(
        _poly_topk_kernel,
        grid=(b, nblk),
        in_specs=[
            pl.BlockSpec((1, 8, g), lambda bi, i: (bi, 0, 0)),
            pl.BlockSpec((1, BLK_P, g), lambda bi, i: (bi, i, 0)),
            pl.BlockSpec((1, BLK_P, g * nray), lambda bi, i: (bi, i, 0)),
            pl.BlockSpec((1, BLK_P, g * nray), lambda bi, i: (bi, i, 0)),
        ],
        out_specs=[
            pl.BlockSpec((1, p, 1), lambda bi, i: (bi, 0, 0)),
            pl.BlockSpec((1, 16, g), lambda bi, i: (bi, 0, 0)),
        ],
        out_shape=[
            jax.ShapeDtypeStruct((b, p, 1), jnp.int32),
            jax.ShapeDtypeStruct((b, 16, g), jnp.int32),
        ],
        scratch_shapes=[pltpu.VMEM((p, g), jnp.float32)],
        compiler_params=pltpu.CompilerParams(
            dimension_semantics=("arbitrary", "arbitrary")),
    )(lt3, cpart,
      poly_preds.reshape(b, p, g * nray), poly_targs.reshape(b, p, g * nray))

    pos_class_targ = pct3[:, :, 0]
    pos_indices = pi3[:, :NUM_SAMPLE, :].reshape(b, NUM_SAMPLE * g)
    return pos_class_targ, pos_indices


# speedup vs baseline: 1.1695x; 1.1250x over previous
import jax
import jax.numpy as jnp
from jax.experimental import pallas as pl
from jax.experimental.pallas import tpu as pltpu

NUM_SAMPLE = 9
ALPHA = 0.25
GAMMA = 2.0
BLK_P = 512


def _cost_topk_kernel(lt_ref, lp_ref, pp_ref, pt_ref, mp_ref, mt_ref,
                      pct_ref, pi_ref, c_ref):
    i = pl.program_id(1)
    nblk = pl.num_programs(1)
    g = lt_ref.shape[2]
    nray = pp_ref.shape[2]
    hw = mp_ref.shape[2]

    lt = lt_ref[0]
    labels_row = lt[0:1, :]

    x = lp_ref[0]
    lp = jax.nn.sigmoid(x)
    neg = (1.0 - ALPHA) * lp ** GAMMA * -jnp.log(1.0 - lp + 1e-08)
    pos = ALPHA * (1.0 - lp) ** GAMMA * -jnp.log(lp + 1e-08)
    diff = pos - neg
    ncls = x.shape[1]
    onehot = (jax.lax.broadcasted_iota(jnp.int32, (ncls, g), 0)
              == labels_row).astype(jnp.float32)
    cc = jnp.dot(diff, onehot, preferred_element_type=jnp.float32,
                 precision=jax.lax.Precision.HIGHEST)

    ppv = pp_ref[0]
    ptv = pt_ref[0]
    lmax = jnp.maximum(ppv, ptv)
    lmin = jnp.minimum(ppv, ptv)
    lcat = jnp.concatenate([lmax, lmin], axis=1)
    iot2 = jax.lax.broadcasted_iota(jnp.int32, (2 * nray, 2), 0)
    sel2 = (iot2 // nray
            == jax.lax.broadcasted_iota(jnp.int32, (2 * nray, 2), 1)
            ).astype(jnp.float32)
    smm = jnp.dot(lcat, sel2, preferred_element_type=jnp.float32,
                  precision=jax.lax.Precision.HIGHEST)
    vm_rows = jnp.log(smm[:, 0:1] / smm[:, 1:2])

    mpv = mp_ref[0]
    mtv = mt_ref[0]
    hw2 = hw // 2
    prod = mpv * mtv
    summ = mpv + mtv
    mcat = jnp.concatenate([prod[:, :hw2] + prod[:, hw2:],
                            summ[:, :hw2] + summ[:, hw2:]], axis=1)
    iotm = jax.lax.broadcasted_iota(jnp.int32, (hw, 2), 0)
    selm = (iotm // hw2
            == jax.lax.broadcasted_iota(jnp.int32, (hw, 2), 1)
            ).astype(jnp.float32)
    ab = jnp.dot(mcat, selm, preferred_element_type=jnp.float32,
                 precision=jax.lax.Precision.HIGHEST)
    dice = (2.0 * ab[:, 0:1] + 1.0) / (ab[:, 1:2] + 1.0)

    c_rows = vm_rows + (1.0 - dice)
    blk_p = lp_ref.shape[1]
    c_ref[pl.ds(i * blk_p, blk_p), :] = c_rows.reshape(blk_p, g) + cc

    @pl.when(i == nblk - 1)
    def _():
        c = c_ref[:, :]
        p = c.shape[0]
        iota_r = jax.lax.broadcasted_iota(jnp.int32, (p, g), 0)
        iota_c = jax.lax.broadcasted_iota(jnp.int32, (p, g), 1)
        cols8 = jax.lax.broadcasted_iota(jnp.int32, (1, g), 1)
        best = jnp.full((p, g), -1, jnp.int32)
        pi_rows = []
        for k in range(NUM_SAMPLE):
            m = jnp.min(c, axis=0, keepdims=True)
            idxk = jnp.min(jnp.where(c == m, iota_r, p),
                           axis=0, keepdims=True)
            match = iota_r == idxk
            enc = jnp.where(match, (k * g + iota_c) * 128 + labels_row, -1)
            best = jnp.maximum(best, enc)
            pi_rows.append(idxk * g + cols8)
            c = jnp.where(match, jnp.float32(jnp.inf), c)
        best1 = jnp.max(best, axis=1, keepdims=True)
        pct_ref[0] = jnp.where(best1 < 0, 80,
                               jnp.bitwise_and(best1, 127)).astype(jnp.int32)
        pi_rows += [jnp.zeros((1, g), jnp.int32)] * (16 - NUM_SAMPLE)
        pi_ref[0] = jnp.concatenate(pi_rows, axis=0)


def kernel(label_targs, label_preds, poly_targs, poly_preds,
           mask_targs, mask_preds, inside_indices):
    b, p, _ = label_preds.shape
    g = label_targs.shape[1]
    nray = poly_targs.shape[-1]
    hw = mask_targs.shape[-1]
    nblk = p // BLK_P
    blk_ni = BLK_P * g

    lt3 = jnp.broadcast_to(label_targs[:, None, :].astype(jnp.int32),
                           (b, 8, g))

    pct3, pi3 = pl.pallas_call(
        _cost_topk_kernel,
        grid=(b, nblk),
        in_specs=[
            pl.BlockSpec((1, 8, g), lambda bi, i: (bi, 0, 0)),
            pl.BlockSpec((1, BLK_P, 80), lambda bi, i: (bi, i, 0)),
            pl.BlockSpec((1, blk_ni, nray), lambda bi, i: (bi, i, 0)),
            pl.BlockSpec((1, blk_ni, nray), lambda bi, i: (bi, i, 0)),
            pl.BlockSpec((1, blk_ni, hw), lambda bi, i: (bi, i, 0)),
            pl.BlockSpec((1, blk_ni, hw), lambda bi, i: (bi, i, 0)),
        ],
        out_specs=[
            pl.BlockSpec((1, p, 1), lambda bi, i: (bi, 0, 0)),
            pl.BlockSpec((1, 16, g), lambda bi, i: (bi, 0, 0)),
        ],
        out_shape=[
            jax.ShapeDtypeStruct((b, p, 1), jnp.int32),
            jax.ShapeDtypeStruct((b, 16, g), jnp.int32),
        ],
        scratch_shapes=[pltpu.VMEM((p, g), jnp.float32)],
        compiler_params=pltpu.CompilerParams(
            dimension_semantics=("arbitrary", "arbitrary")),
    )(lt3, label_preds, poly_preds, poly_targs, mask_preds, mask_targs)

    pos_class_targ = pct3[:, :, 0]
    pos_indices = pi3[:, :NUM_SAMPLE, :].reshape(b, NUM_SAMPLE * g)
    return pos_class_targ, pos_indices
